# baseline (speedup 1.0000x reference)
import jax
import jax.numpy as jnp
from jax.experimental import pallas as pl

_TILE = 1024


def _fused_body(x_ref, wlab_ref, blab_ref, wenc_ref, benc_ref, wc_ref, bc_ref,
                out_ref):
    x = x_ref[...]
    lab = jnp.dot(x, wlab_ref[...], preferred_element_type=jnp.float32)
    lab = lab + blab_ref[...]
    y = jnp.argmax(lab, axis=-1)
    xb = x.astype(jnp.bfloat16)
    z = jnp.dot(xb, wenc_ref[...], preferred_element_type=jnp.float32)
    z = z + benc_ref[...]
    logits = jnp.dot(z.astype(jnp.bfloat16), wc_ref[...],
                     preferred_element_type=jnp.float32)
    logits = logits + bc_ref[...]
    expert_of_col = jax.lax.broadcasted_iota(jnp.int32, logits.shape, 1) // 16
    out_ref[...] = jnp.where(expert_of_col == y[:, None], logits, 0.0)


def kernel(x, W_lab, b_lab, W_enc, b_enc, W_clf, b_clf):
    N, D = x.shape
    E, H, S = W_clf.shape
    Wc = jnp.transpose(W_clf, (1, 0, 2)).reshape(H, E * S)
    bc = b_clf.reshape(1, E * S)
    grid = (N // _TILE,)
    return pl.pallas_call(
        _fused_body,
        grid=grid,
        in_specs=[
            pl.BlockSpec((_TILE, D), lambda i: (i, 0)),
            pl.BlockSpec((D, E), lambda i: (0, 0)),
            pl.BlockSpec((1, E), lambda i: (0, 0)),
            pl.BlockSpec((D, H), lambda i: (0, 0)),
            pl.BlockSpec((1, H), lambda i: (0, 0)),
            pl.BlockSpec((H, E * S), lambda i: (0, 0)),
            pl.BlockSpec((1, E * S), lambda i: (0, 0)),
        ],
        out_specs=pl.BlockSpec((_TILE, E * S), lambda i: (i, 0)),
        out_shape=jax.ShapeDtypeStruct((N, E * S), x.dtype),
    )(x, W_lab, b_lab[None, :], W_enc.astype(jnp.bfloat16), b_enc[None, :],
      Wc.astype(jnp.bfloat16), bc)


# speedup vs baseline: 1.2336x; 1.2336x over previous
import jax
import jax.numpy as jnp
from jax.experimental import pallas as pl

_TILE = 2048


def _fused_body(x_ref, wlab_ref, blab_ref, wenc_ref, benc_ref, wc_ref, bc_ref,
                out_ref):
    x = x_ref[...]
    lab = jnp.dot(x, wlab_ref[...], preferred_element_type=jnp.float32)
    lab = lab + blab_ref[...]
    y = jnp.argmax(lab, axis=-1)
    z = jnp.dot(x, wenc_ref[...], preferred_element_type=jnp.float32)
    z = z + benc_ref[...]
    logits = jnp.dot(z, wc_ref[...], preferred_element_type=jnp.float32)
    logits = logits + bc_ref[...]
    expert_of_col = jax.lax.broadcasted_iota(jnp.int32, logits.shape, 1) // 16
    out_ref[...] = jnp.where(expert_of_col == y[:, None], logits, 0.0)


def kernel(x, W_lab, b_lab, W_enc, b_enc, W_clf, b_clf):
    N, D = x.shape
    E, H, S = W_clf.shape
    Wc = jnp.transpose(W_clf, (1, 0, 2)).reshape(H, E * S)
    bc = b_clf.reshape(1, E * S)
    grid = (N // _TILE,)
    return pl.pallas_call(
        _fused_body,
        grid=grid,
        in_specs=[
            pl.BlockSpec((_TILE, D), lambda i: (i, 0)),
            pl.BlockSpec((D, E), lambda i: (0, 0)),
            pl.BlockSpec((1, E), lambda i: (0, 0)),
            pl.BlockSpec((D, H), lambda i: (0, 0)),
            pl.BlockSpec((1, H), lambda i: (0, 0)),
            pl.BlockSpec((H, E * S), lambda i: (0, 0)),
            pl.BlockSpec((1, E * S), lambda i: (0, 0)),
        ],
        out_specs=pl.BlockSpec((_TILE, E * S), lambda i: (i, 0)),
        out_shape=jax.ShapeDtypeStruct((N, E * S), x.dtype),
    )(x, W_lab, b_lab[None, :], W_enc, b_enc[None, :], Wc, bc)


# speedup vs baseline: 1.3346x; 1.0819x over previous
import jax
import jax.numpy as jnp
from jax.experimental import pallas as pl

_TILE = 4096


def _fused_body(x_ref, wlab_ref, blab_ref, wenc_ref, benc_ref, wc_ref, bc_ref,
                out_ref):
    x = x_ref[...]
    lab = jnp.dot(x, wlab_ref[...], preferred_element_type=jnp.float32)
    lab = lab + blab_ref[...]
    y = jnp.argmax(lab, axis=-1)
    z = jnp.dot(x, wenc_ref[...], preferred_element_type=jnp.float32)
    z = z + benc_ref[...]
    logits = jnp.dot(z, wc_ref[...], preferred_element_type=jnp.float32)
    logits = logits + bc_ref[...]
    expert_of_col = jax.lax.broadcasted_iota(jnp.int32, logits.shape, 1) // 16
    out_ref[...] = jnp.where(expert_of_col == y[:, None], logits, 0.0)


def kernel(x, W_lab, b_lab, W_enc, b_enc, W_clf, b_clf):
    N, D = x.shape
    E, H, S = W_clf.shape
    Wc = jnp.transpose(W_clf, (1, 0, 2)).reshape(H, E * S)
    bc = b_clf.reshape(1, E * S)
    grid = (N // _TILE,)
    return pl.pallas_call(
        _fused_body,
        grid=grid,
        in_specs=[
            pl.BlockSpec((_TILE, D), lambda i: (i, 0)),
            pl.BlockSpec((D, E), lambda i: (0, 0)),
            pl.BlockSpec((1, E), lambda i: (0, 0)),
            pl.BlockSpec((D, H), lambda i: (0, 0)),
            pl.BlockSpec((1, H), lambda i: (0, 0)),
            pl.BlockSpec((H, E * S), lambda i: (0, 0)),
            pl.BlockSpec((1, E * S), lambda i: (0, 0)),
        ],
        out_specs=pl.BlockSpec((_TILE, E * S), lambda i: (i, 0)),
        out_shape=jax.ShapeDtypeStruct((N, E * S), x.dtype),
    )(x, W_lab, b_lab[None, :], W_enc, b_enc[None, :], Wc, bc)
